# baseline (speedup 1.0000x reference)
import jax
import jax.numpy as jnp
import numpy as np
from jax import lax
from jax.experimental import pallas as pl
from jax.experimental.pallas import tpu as pltpu
from jax.experimental.pallas import tpu_sc as plsc

NUM_ATOMS = 4608
NUM_IN_DEG = 512
NUM_OUT_DEG = 512
H = 768
HW = H // 2
B = 256
N = 128
F = 9
IPN = F + 2
NC = 2
NS = 16
NW = NC * NS
GPW = B // NW
C = 8
KPG = N // C
IPC = C * IPN
NBLK = H // 32

_BLK = np.arange(32).reshape(2, 16).T.reshape(-1)
_PERM = (np.arange(0, H, 32)[:, None] + _BLK[None, :]).reshape(-1)
_MASK_HI = np.int32(-65536)


def _sum_chunk(buf, ost):
    def blk(k, carry):
        w0 = 16 * k
        c0 = 32 * k
        for i in range(C):
            v = buf[i * IPN, pl.ds(w0, 16)]
            hi = plsc.bitcast(v & _MASK_HI, jnp.float32)
            lo = plsc.bitcast(v << 16, jnp.float32)
            for j in range(1, IPN):
                v = buf[i * IPN + j, pl.ds(w0, 16)]
                hi = hi + plsc.bitcast(v & _MASK_HI, jnp.float32)
                lo = lo + plsc.bitcast(v << 16, jnp.float32)
            ost[i, pl.ds(c0, 16)] = lo
            ost[i, pl.ds(c0 + 16, 16)] = hi
        return carry
    lax.fori_loop(0, NBLK, blk, 0, unroll=False)


def _graph_node_feature_kernel(table_hbm, idx_hbm, tok_hbm, out_hbm,
                               idx_v, buf0, buf1, ost0, ost1, tok_v,
                               sg0, sg1, ss0, ss1):
    wid = lax.axis_index("s") * NC + lax.axis_index("c")

    pltpu.sync_copy(tok_hbm, tok_v)

    def graph_body(g, carry):
        gid = wid * GPW + g
        pltpu.sync_copy(idx_hbm.at[gid], idx_v)
        pltpu.sync_copy(tok_v, out_hbm.at[gid * (N + 1)])

        pltpu.async_copy(table_hbm.at[idx_v.at[0]], buf0, sg0)

        def pair(t, c2):
            k0 = 2 * t
            row0 = gid * (N + 1) + 1 + C * k0

            pltpu.async_copy(table_hbm.at[idx_v.at[k0 + 1]], buf1, sg1)

            pltpu.make_async_copy(table_hbm.at[idx_v.at[0]], buf0, sg0).wait()

            @pl.when(t > 0)
            def _wait_prev_store0():
                pltpu.make_async_copy(ost0, out_hbm.at[pl.ds(0, C)], ss0).wait()

            _sum_chunk(buf0, ost0)
            pltpu.async_copy(ost0, out_hbm.at[pl.ds(row0, C)], ss0)

            @pl.when(t < KPG // 2 - 1)
            def _fire_next():
                pltpu.async_copy(table_hbm.at[idx_v.at[k0 + 2]], buf0, sg0)

            pltpu.make_async_copy(table_hbm.at[idx_v.at[0]], buf1, sg1).wait()

            @pl.when(t > 0)
            def _wait_prev_store1():
                pltpu.make_async_copy(ost1, out_hbm.at[pl.ds(0, C)], ss1).wait()

            _sum_chunk(buf1, ost1)
            pltpu.async_copy(ost1, out_hbm.at[pl.ds(row0 + C, C)], ss1)
            return c2

        lax.fori_loop(0, KPG // 2, pair, 0, unroll=False)

        pltpu.make_async_copy(ost0, out_hbm.at[pl.ds(0, C)], ss0).wait()
        pltpu.make_async_copy(ost1, out_hbm.at[pl.ds(0, C)], ss1).wait()
        return carry

    lax.fori_loop(0, GPW, graph_body, 0, unroll=False)


@jax.jit
def _run(table, idx, tok):
    mesh = plsc.VectorSubcoreMesh(core_axis_name="c", subcore_axis_name="s")
    return pl.kernel(
        _graph_node_feature_kernel,
        out_type=jax.ShapeDtypeStruct((B * (N + 1), H), jnp.float32),
        mesh=mesh,
        scratch_types=[
            pltpu.VMEM((KPG, IPC), jnp.int32),
            pltpu.VMEM((IPC, HW), jnp.int32),
            pltpu.VMEM((IPC, HW), jnp.int32),
            pltpu.VMEM((C, H), jnp.float32),
            pltpu.VMEM((C, H), jnp.float32),
            pltpu.VMEM((H,), jnp.float32),
            pltpu.SemaphoreType.DMA,
            pltpu.SemaphoreType.DMA,
            pltpu.SemaphoreType.DMA,
            pltpu.SemaphoreType.DMA,
        ],
        compiler_params=pltpu.CompilerParams(
            use_tc_tiling_on_sc=False, needs_layout_passes=False),
    )(table, idx, tok)


def kernel(x, in_degree, out_degree, atom_table, in_table, out_table, graph_token):
    x = x.astype(jnp.int32)
    in_degree = in_degree.astype(jnp.int32)
    out_degree = out_degree.astype(jnp.int32)
    idx = jnp.concatenate(
        [
            x,
            (in_degree + (NUM_ATOMS + 1))[..., None],
            (out_degree + (NUM_ATOMS + 1 + NUM_IN_DEG))[..., None],
        ],
        axis=-1,
    ).reshape(B, KPG, IPC)
    table = jnp.concatenate([atom_table, in_table, out_table], axis=0)
    packed = lax.bitcast_convert_type(
        table.astype(jnp.bfloat16)[:, _PERM].reshape(-1, HW, 2), jnp.int32)
    out = _run(packed, idx, graph_token.reshape(H))
    return out.reshape(B, N + 1, H)


# speedup vs baseline: 1.0280x; 1.0280x over previous
import jax
import jax.numpy as jnp
import numpy as np
from jax import lax
from jax.experimental import pallas as pl
from jax.experimental.pallas import tpu as pltpu
from jax.experimental.pallas import tpu_sc as plsc

NUM_ATOMS = 4608
NUM_IN_DEG = 512
NUM_OUT_DEG = 512
H = 768
HW = H // 2
B = 256
N = 128
F = 9
IPN = F + 2
NC = 2
NS = 16
NW = NC * NS
GPW = B // NW
C = 4
KPG = N // C
IPC = C * IPN
NBLK = H // 32
NBUF = 4

_MASK_HI = np.int32(-65536)


def _sum_chunk(buf, ost, ev2):
    def blk(k, carry):
        w0 = 16 * k
        for i in range(C):
            v = buf[i * IPN, pl.ds(w0, 16)]
            hi = plsc.bitcast(v & _MASK_HI, jnp.float32)
            lo = plsc.bitcast(v << 16, jnp.float32)
            for j in range(1, IPN):
                v = buf[i * IPN + j, pl.ds(w0, 16)]
                hi = hi + plsc.bitcast(v & _MASK_HI, jnp.float32)
                lo = lo + plsc.bitcast(v << 16, jnp.float32)
            base = ev2 + (i * H + 32 * k)
            plsc.store_scatter(ost, [base], lo)
            plsc.store_scatter(ost, [base + 1], hi)
        return carry
    lax.fori_loop(0, NBLK, blk, 0, unroll=False)


def _graph_node_feature_kernel(table_hbm, idx_hbm, tok_hbm, out_hbm,
                               idx_v, bufs, osts, tok_v, sg, ss):
    wid = lax.axis_index("s") * NC + lax.axis_index("c")
    ev2 = jax.lax.iota(jnp.int32, 16) * 2

    pltpu.sync_copy(tok_hbm, tok_v)

    def graph_body(g, carry):
        gid = wid * GPW + g
        pltpu.sync_copy(idx_hbm.at[gid], idx_v)
        pltpu.sync_copy(tok_v, out_hbm.at[pl.ds(gid * (N + 1) * H, H)])

        for b in range(NBUF - 1):
            pltpu.async_copy(table_hbm.at[idx_v.at[b]], bufs[b], sg[b])

        def ring(t, c2):
            for b in range(NBUF):
                k = NBUF * t + b

                @pl.when(k + NBUF - 1 < KPG)
                def _fire():
                    pltpu.async_copy(
                        table_hbm.at[idx_v.at[k + NBUF - 1]],
                        bufs[(b + NBUF - 1) % NBUF],
                        sg[(b + NBUF - 1) % NBUF])

                pltpu.make_async_copy(
                    table_hbm.at[idx_v.at[0]], bufs[b], sg[b]).wait()

                @pl.when(t > 0)
                def _wait_prev_store():
                    pltpu.make_async_copy(
                        osts[b], out_hbm.at[pl.ds(0, C * H)], ss[b]).wait()

                _sum_chunk(bufs[b], osts[b], ev2)
                row0 = (gid * (N + 1) + 1 + C * k) * H
                pltpu.async_copy(osts[b], out_hbm.at[pl.ds(row0, C * H)], ss[b])
            return c2

        lax.fori_loop(0, KPG // NBUF, ring, 0, unroll=False)

        for b in range(NBUF):
            pltpu.make_async_copy(
                osts[b], out_hbm.at[pl.ds(0, C * H)], ss[b]).wait()
        return carry

    lax.fori_loop(0, GPW, graph_body, 0, unroll=False)


@jax.jit
def _run(table, idx, tok):
    mesh = plsc.VectorSubcoreMesh(core_axis_name="c", subcore_axis_name="s")
    return pl.kernel(
        _graph_node_feature_kernel,
        out_type=jax.ShapeDtypeStruct((B * (N + 1) * H,), jnp.float32),
        mesh=mesh,
        scratch_types=[
            pltpu.VMEM((KPG, IPC), jnp.int32),
            [pltpu.VMEM((IPC, HW), jnp.int32) for _ in range(NBUF)],
            [pltpu.VMEM((C * H,), jnp.float32) for _ in range(NBUF)],
            pltpu.VMEM((H,), jnp.float32),
            [pltpu.SemaphoreType.DMA for _ in range(NBUF)],
            [pltpu.SemaphoreType.DMA for _ in range(NBUF)],
        ],
        compiler_params=pltpu.CompilerParams(
            use_tc_tiling_on_sc=False, needs_layout_passes=False),
    )(table, idx, tok)


def kernel(x, in_degree, out_degree, atom_table, in_table, out_table, graph_token):
    x = x.astype(jnp.int32)
    in_degree = in_degree.astype(jnp.int32)
    out_degree = out_degree.astype(jnp.int32)
    idx = jnp.concatenate(
        [
            x,
            (in_degree + (NUM_ATOMS + 1))[..., None],
            (out_degree + (NUM_ATOMS + 1 + NUM_IN_DEG))[..., None],
        ],
        axis=-1,
    ).reshape(B, KPG, IPC)
    table = jnp.concatenate([atom_table, in_table, out_table], axis=0)
    packed = lax.bitcast_convert_type(
        table.astype(jnp.bfloat16).reshape(-1, HW, 2), jnp.int32)
    out = _run(packed, idx, graph_token.reshape(H))
    return out.reshape(B, N + 1, H)
